# Initial kernel scaffold; baseline (speedup 1.0000x reference)
import jax
import jax.numpy as jnp
from jax.experimental import pallas as pl


def kernel(node, edge, n2n, e2n, idx_node, idx_edge, W_en, b_en, W_ee, b_ee, Wn, bn, We, be, W_fc, b_fc):
    raise NotImplementedError("write your pallas kernel here")


# speedup vs baseline: 1.0195x; 1.0195x over previous
import functools

import jax
import jax.numpy as jnp
from jax.experimental import pallas as pl
from jax.experimental.pallas import tpu as pltpu

N = 10000
E = 320000
G = 256
D = 128

_POOL_R = 1024
_N_PAD = 10240


def _pool_body(idx_ref, x_ref, o_ref, acc, cnt):
    step = pl.program_id(0)

    @pl.when(step == 0)
    def _init():
        acc[...] = jnp.zeros_like(acc)
        cnt[...] = jnp.zeros_like(cnt)

    idx = idx_ref[0, 0, :]
    onehot = (jax.lax.broadcasted_iota(jnp.int32, (G, _POOL_R), 0)
              == idx[None, :]).astype(jnp.float32)
    acc[...] += jnp.dot(onehot, x_ref[...],
                        preferred_element_type=jnp.float32)
    cnt[...] += jnp.sum(onehot, axis=1, keepdims=True)

    @pl.when(step == pl.num_programs(0) - 1)
    def _fini():
        o_ref[...] = acc[...] / jnp.clip(cnt[...], 1.0, None)


def _pool(x, idx):
    xp = jnp.zeros((_N_PAD, D), jnp.float32).at[:N].set(x)
    ip = jnp.full((_N_PAD,), G, jnp.int32).at[:N].set(idx.astype(jnp.int32))
    ip = ip.reshape(_N_PAD // _POOL_R, 1, _POOL_R)
    grid = _N_PAD // _POOL_R
    return pl.pallas_call(
        _pool_body,
        grid=(grid,),
        in_specs=[
            pl.BlockSpec((1, 1, _POOL_R), lambda i: (i, i * 0, i * 0)),
            pl.BlockSpec((_POOL_R, D), lambda i: (i, i * 0)),
        ],
        out_specs=pl.BlockSpec((G, D), lambda i: (i * 0, i * 0)),
        out_shape=jax.ShapeDtypeStruct((G, D), jnp.float32),
        scratch_shapes=[
            pltpu.VMEM((G, D), jnp.float32),
            pltpu.VMEM((G, 1), jnp.float32),
        ],
    )(ip, xp)


@functools.partial(jax.jit, static_argnums=())
def kernel(node, edge, n2n, e2n, idx_node, idx_edge, W_en, b_en, W_ee, b_ee,
           Wn, bn, We, be, W_fc, b_fc):
    del n2n, idx_edge, W_fc, b_fc
    src = e2n[0].astype(jnp.int32)
    dst = e2n[1].astype(jnp.int32)
    xn = node @ W_en + b_en
    xe = edge @ W_ee + b_ee
    L = Wn.shape[0]
    for i in range(L):
        msg = xe + jnp.take(xn, src, axis=0)
        agg = jax.ops.segment_sum(msg, dst, num_segments=N)
        xn = jax.nn.relu((xn + agg) @ Wn[i] + bn[i])
        if i < L - 1:
            xe = jax.nn.relu(
                (xe + jnp.take(xn, src, axis=0) + jnp.take(xn, dst, axis=0))
                @ We[i] + be[i])
    return _pool(xn, idx_node)


# speedup vs baseline: 1.4147x; 1.3876x over previous
import functools

import jax
import jax.numpy as jnp
from jax import lax
from jax.experimental import pallas as pl
from jax.experimental.pallas import tpu as pltpu
from jax.experimental.pallas import tpu_sc as plsc

N = 10000
E = 320000
G = 256
D = 128
DH = 64
NC = 2
NS = 16
EPS = E // NS
CH = 400
NCHUNK = EPS // CH
ROWS_PS = N // NS

_POOL_R = 1024
_N_PAD = 10240


EPC = E // NC
EPSUB = EPC // NS
CHE = 80
NCHE = EPSUB // CHE


def _sc_agg_body(xe_ref, src_ref, dst_ref, xn_ref, part_ref,
                 acc, src_v, dst_v, data_v, tmp_v, gsem):
    i32 = jnp.int32
    c = lax.axis_index("c").astype(i32)
    s = lax.axis_index("s").astype(i32)

    def _zero(i, _):
        for j in range(D // 16):
            tmp_v[i, pl.ds(j * 16, 16)] = jnp.zeros((16,), jnp.float32)
        return 0
    lax.fori_loop(jnp.int32(0), jnp.int32(CHE), _zero, 0)
    z0 = s * i32(625)
    for off in range(0, 560, CHE):
        pltpu.sync_copy(tmp_v, acc.at[pl.ds(z0 + i32(off), CHE)])
    pltpu.sync_copy(tmp_v.at[pl.ds(0, 65)], acc.at[pl.ds(z0 + i32(560), 65)])
    plsc.subcore_barrier()

    def _chunk(k, _):
        base = pl.multiple_of(c * i32(EPC) + s * i32(EPSUB) + k * i32(CHE), 8)
        pltpu.sync_copy(src_ref.at[pl.ds(base, CHE)], src_v)
        pltpu.sync_copy(dst_ref.at[pl.ds(base, CHE)], dst_v)
        pltpu.sync_copy(xe_ref.at[pl.ds(base, CHE)], data_v)
        pltpu.async_copy(xn_ref.at[src_v], tmp_v, gsem).wait()

        def _addrow(r, _):
            for j in range(D // 16):
                sl = pl.ds(j * 16, 16)
                tmp_v[r, sl] = data_v[r, sl] + tmp_v[r, sl]
            return 0
        lax.fori_loop(jnp.int32(0), jnp.int32(CHE), _addrow, 0)

        pltpu.sync_copy(tmp_v, acc.at[dst_v], add=True)
        return 0
    lax.fori_loop(jnp.int32(0), jnp.int32(NCHE), _chunk, 0)

    plsc.subcore_barrier()

    @pl.when(s < 10)
    def _writeout():
        r0 = pl.multiple_of(s * i32(1000), 8)
        pltpu.sync_copy(acc.at[pl.ds(r0, 1000)],
                        part_ref.at[c, pl.ds(r0, 1000)])


def _sc_agg(xe, src, dst, xn):
    mesh = plsc.VectorSubcoreMesh(core_axis_name="c", subcore_axis_name="s")
    f = pl.kernel(
        _sc_agg_body,
        out_type=jax.ShapeDtypeStruct((NC, N, D), jnp.float32),
        mesh=mesh,
        scratch_types=[
            pltpu.VMEM_SHARED((N, D), jnp.float32),
            pltpu.VMEM((CHE,), jnp.int32),
            pltpu.VMEM((CHE,), jnp.int32),
            pltpu.VMEM((CHE, D), jnp.float32),
            pltpu.VMEM((CHE, D), jnp.float32),
            pltpu.SemaphoreType.DMA,
        ],
    )
    return f(xe, src, dst, xn)


def _pool_body(idx_ref, x_ref, o_ref, acc, cnt):
    step = pl.program_id(0)

    @pl.when(step == 0)
    def _init():
        acc[...] = jnp.zeros_like(acc)
        cnt[...] = jnp.zeros_like(cnt)

    idx = idx_ref[0, 0, :]
    onehot = (jax.lax.broadcasted_iota(jnp.int32, (G, _POOL_R), 0)
              == idx[None, :]).astype(jnp.float32)
    acc[...] += jnp.dot(onehot, x_ref[...],
                        preferred_element_type=jnp.float32)
    cnt[...] += jnp.sum(onehot, axis=1, keepdims=True)

    @pl.when(step == pl.num_programs(0) - 1)
    def _fini():
        o_ref[...] = acc[...] / jnp.clip(cnt[...], 1.0, None)


def _pool(x, idx):
    xp = jnp.zeros((_N_PAD, D), jnp.float32).at[:N].set(x)
    ip = jnp.full((_N_PAD,), G, jnp.int32).at[:N].set(idx.astype(jnp.int32))
    ip = ip.reshape(_N_PAD // _POOL_R, 1, _POOL_R)
    grid = _N_PAD // _POOL_R
    return pl.pallas_call(
        _pool_body,
        grid=(grid,),
        in_specs=[
            pl.BlockSpec((1, 1, _POOL_R), lambda i: (i, i * 0, i * 0)),
            pl.BlockSpec((_POOL_R, D), lambda i: (i, i * 0)),
        ],
        out_specs=pl.BlockSpec((G, D), lambda i: (i * 0, i * 0)),
        out_shape=jax.ShapeDtypeStruct((G, D), jnp.float32),
        scratch_shapes=[
            pltpu.VMEM((G, D), jnp.float32),
            pltpu.VMEM((G, 1), jnp.float32),
        ],
    )(ip, xp)


@functools.partial(jax.jit, static_argnums=())
def kernel(node, edge, n2n, e2n, idx_node, idx_edge, W_en, b_en, W_ee, b_ee,
           Wn, bn, We, be, W_fc, b_fc):
    del n2n, idx_edge, W_fc, b_fc
    src = e2n[0].astype(jnp.int32)
    dst = e2n[1].astype(jnp.int32)
    xn = node @ W_en + b_en
    xe = edge @ W_ee + b_ee
    L = Wn.shape[0]
    for i in range(L):
        part = _sc_agg(xe, src, dst, xn)
        agg = part[0] + part[1]
        xn = jax.nn.relu((xn + agg) @ Wn[i] + bn[i])
        if i < L - 1:
            xe = jax.nn.relu(
                (xe + jnp.take(xn, src, axis=0) + jnp.take(xn, dst, axis=0))
                @ We[i] + be[i])
    return _pool(xn, idx_node)
